# baseline (device time: 81409 ns/iter reference)
import jax
import jax.numpy as jnp
from jax import lax
from jax.experimental import pallas as pl
from jax.experimental.pallas import tpu as pltpu

N_DEV = 32
LOG2 = 5
B, SQ, HQ, DH = 2, 256, 4, 64
D_MODEL = 512
HD = HQ * DH
SKV_LOC = 256
PACK = HD + HQ

_DevT = getattr(pl, "DeviceIdType", None) or pltpu.DeviceIdType


def kernel(x, Wq, K_ext, V_ext, Wo):
    K2 = K_ext.reshape(B, SKV_LOC, HD)
    V2 = V_ext.reshape(B, SKV_LOC, HD)

    def body(x_ref, wq_ref, k_ref, v_ref, wo_ref, out_ref,
             part_ref, recv_ref, send_sems, recv_sems):
        me = lax.axis_index("i")

        bar = pltpu.get_barrier_semaphore()
        for k in range(LOG2):
            pl.semaphore_signal(bar, inc=1, device_id=(me ^ (1 << k),),
                                device_id_type=_DevT.MESH)
        pl.semaphore_wait(bar, LOG2)

        wq = wq_ref[:, :].astype(jnp.bfloat16)
        rows = lax.broadcasted_iota(jnp.int32, (SQ, SKV_LOC), 0) // 64
        cols = lax.broadcasted_iota(jnp.int32, (SQ, SKV_LOC), 1) // 64
        mask = rows == cols
        for b in range(B):
            q_b = jnp.dot(x_ref[b].astype(jnp.bfloat16), wq,
                          preferred_element_type=jnp.float32)
            q16 = q_b.astype(jnp.bfloat16)
            for h in range(HQ):
                q_bh = q16[:, h * DH:(h + 1) * DH]
                k_bh = k_ref[b][:, h * DH:(h + 1) * DH].astype(jnp.bfloat16)
                v_bh = v_ref[b][:, h * DH:(h + 1) * DH].astype(jnp.bfloat16)
                s = lax.dot_general(q_bh, k_bh, (((1,), (1,)), ((), ())),
                                    preferred_element_type=jnp.float32)
                p = jnp.where(mask, jnp.exp(s * 0.125), 0.0)
                l = jnp.sum(p, axis=1, keepdims=True)
                acc = jnp.dot(p.astype(jnp.bfloat16), v_bh,
                              preferred_element_type=jnp.float32)
                part_ref[b, :, h * DH:(h + 1) * DH] = acc
                part_ref[b, :, HD + h:HD + h + 1] = l

        for k in range(LOG2):
            partner = me ^ (1 << k)
            rdma = pltpu.make_async_remote_copy(
                src_ref=part_ref,
                dst_ref=recv_ref.at[k],
                send_sem=send_sems.at[k],
                recv_sem=recv_sems.at[k],
                device_id=(partner,),
                device_id_type=_DevT.MESH,
            )
            rdma.start()
            rdma.wait()
            part_ref[...] = part_ref[...] + recv_ref[k]

        wo = wo_ref[:, :].astype(jnp.bfloat16)
        for b in range(B):
            denom = jnp.concatenate(
                [jnp.broadcast_to(part_ref[b, :, HD + h:HD + h + 1], (SQ, DH))
                 for h in range(HQ)], axis=1)
            ctx = part_ref[b, :, 0:HD] / denom
            out_ref[b] = jnp.dot(ctx.astype(jnp.bfloat16), wo,
                                 preferred_element_type=jnp.float32)

    return pl.pallas_call(
        body,
        out_shape=jax.ShapeDtypeStruct((B, SQ, D_MODEL), jnp.float32),
        in_specs=[pl.BlockSpec(memory_space=pltpu.VMEM)] * 5,
        out_specs=pl.BlockSpec(memory_space=pltpu.VMEM),
        scratch_shapes=[
            pltpu.VMEM((B, SQ, PACK), jnp.float32),
            pltpu.VMEM((LOG2, B, SQ, PACK), jnp.float32),
            pltpu.SemaphoreType.DMA((LOG2,)),
            pltpu.SemaphoreType.DMA((LOG2,)),
        ],
        compiler_params=pltpu.CompilerParams(collective_id=0),
    )(x, Wq, K2, V2, Wo)


# device time: 52226 ns/iter; 1.5588x vs baseline; 1.5588x over previous
import jax
import jax.numpy as jnp
from jax import lax
from jax.experimental import pallas as pl
from jax.experimental.pallas import tpu as pltpu

N_DEV = 32
LOG2 = 5
B, SQ, HQ, DH = 2, 256, 4, 64
D_MODEL = 512
HD = HQ * DH
SKV_LOC = 256
PACK = HD + HQ
WIRE = 384

_DevT = getattr(pl, "DeviceIdType", None) or pltpu.DeviceIdType


def kernel(x, Wq, K_ext, V_ext, Wo):
    K2 = K_ext.reshape(B, SKV_LOC, HD)
    V2 = V_ext.reshape(B, SKV_LOC, HD)

    def body(x_ref, wq_ref, k_ref, v_ref, wo_ref, out_ref,
             part_ref, send_ref, recv_ref, send_sems, recv_sems):
        me = lax.axis_index("i")

        bar = pltpu.get_barrier_semaphore()
        for k in range(LOG2):
            pl.semaphore_signal(bar, inc=1, device_id=(me ^ (1 << k),),
                                device_id_type=_DevT.MESH)
        pl.semaphore_wait(bar, LOG2)

        wq = wq_ref[:, :].astype(jnp.bfloat16)
        rows = lax.broadcasted_iota(jnp.int32, (SQ, SKV_LOC), 0) // 64
        cols = lax.broadcasted_iota(jnp.int32, (SQ, SKV_LOC), 1) // 64
        mask = rows == cols
        for b in range(B):
            q_b = jnp.dot(x_ref[b].astype(jnp.bfloat16), wq,
                          preferred_element_type=jnp.float32)
            q16 = q_b.astype(jnp.bfloat16)
            for h in range(HQ):
                q_bh = q16[:, h * DH:(h + 1) * DH]
                k_bh = k_ref[b][:, h * DH:(h + 1) * DH].astype(jnp.bfloat16)
                v_bh = v_ref[b][:, h * DH:(h + 1) * DH].astype(jnp.bfloat16)
                s = lax.dot_general(q_bh, k_bh, (((1,), (1,)), ((), ())),
                                    preferred_element_type=jnp.float32)
                p = jnp.where(mask, jnp.exp(s * 0.125), 0.0)
                l = jnp.sum(p, axis=1, keepdims=True)
                acc = jnp.dot(p.astype(jnp.bfloat16), v_bh,
                              preferred_element_type=jnp.float32)
                part_ref[b, :, h * DH:(h + 1) * DH] = acc
                part_ref[b, :, HD + h:HD + h + 1] = l

        for k in range(LOG2):
            partner = me ^ (1 << k)
            send_ref[:, :, 0:PACK] = part_ref[...].astype(jnp.bfloat16)
            rdma = pltpu.make_async_remote_copy(
                src_ref=send_ref,
                dst_ref=recv_ref.at[k],
                send_sem=send_sems.at[k],
                recv_sem=recv_sems.at[k],
                device_id=(partner,),
                device_id_type=_DevT.MESH,
            )
            rdma.start()
            rdma.wait()
            part_ref[...] = (part_ref[...] +
                             recv_ref[k][:, :, 0:PACK].astype(jnp.float32))

        wo = wo_ref[:, :].astype(jnp.bfloat16)
        for b in range(B):
            denom = jnp.concatenate(
                [jnp.broadcast_to(part_ref[b, :, HD + h:HD + h + 1], (SQ, DH))
                 for h in range(HQ)], axis=1)
            ctx = part_ref[b, :, 0:HD] / denom
            out_ref[b] = jnp.dot(ctx.astype(jnp.bfloat16), wo,
                                 preferred_element_type=jnp.float32)

    return pl.pallas_call(
        body,
        out_shape=jax.ShapeDtypeStruct((B, SQ, D_MODEL), jnp.float32),
        in_specs=[pl.BlockSpec(memory_space=pltpu.VMEM)] * 5,
        out_specs=pl.BlockSpec(memory_space=pltpu.VMEM),
        scratch_shapes=[
            pltpu.VMEM((B, SQ, PACK), jnp.float32),
            pltpu.VMEM((B, SQ, WIRE), jnp.bfloat16),
            pltpu.VMEM((LOG2, B, SQ, WIRE), jnp.bfloat16),
            pltpu.SemaphoreType.DMA((LOG2,)),
            pltpu.SemaphoreType.DMA((LOG2,)),
        ],
        compiler_params=pltpu.CompilerParams(collective_id=0),
    )(x, Wq, K2, V2, Wo)


# device time: 43509 ns/iter; 1.8711x vs baseline; 1.2003x over previous
import jax
import jax.numpy as jnp
from jax import lax
from jax.experimental import pallas as pl
from jax.experimental.pallas import tpu as pltpu

N_DEV = 32
LOG2 = 5
B, SQ, HQ, DH = 2, 256, 4, 64
D_MODEL = 512
HD = HQ * DH
SKV_LOC = 256
ROWS = SQ + 16

_DevT = getattr(pl, "DeviceIdType", None) or pltpu.DeviceIdType


def kernel(x, Wq, K_ext, V_ext, Wo):
    K2 = K_ext.reshape(B, SKV_LOC, HD)
    V2 = V_ext.reshape(B, SKV_LOC, HD)

    def body(x_ref, wq_ref, k_ref, v_ref, wo_ref, out_ref,
             part_ref, send_ref, recv_ref, send_sems, recv_sems):
        me = lax.axis_index("i")

        bar = pltpu.get_barrier_semaphore()
        for k in range(LOG2):
            pl.semaphore_signal(bar, inc=1, device_id=(me ^ (1 << k),),
                                device_id_type=_DevT.MESH)
        pl.semaphore_wait(bar, LOG2)

        wq = wq_ref[:, :].astype(jnp.bfloat16)
        rows = lax.broadcasted_iota(jnp.int32, (SQ, SKV_LOC), 0) // 64
        cols = lax.broadcasted_iota(jnp.int32, (SQ, SKV_LOC), 1) // 64
        mask = rows == cols
        ones8 = jnp.ones((8, SQ), jnp.bfloat16)
        for b in range(B):
            part_ref[b, SQ + HQ:ROWS, :] = jnp.zeros((ROWS - SQ - HQ, HD),
                                                     jnp.float32)
            q_b = jnp.dot(x_ref[b].astype(jnp.bfloat16), wq,
                          preferred_element_type=jnp.float32)
            q16 = q_b.astype(jnp.bfloat16)
            for h in range(HQ):
                q_bh = q16[:, h * DH:(h + 1) * DH]
                k_bh = k_ref[b][:, h * DH:(h + 1) * DH].astype(jnp.bfloat16)
                v_bh = v_ref[b][:, h * DH:(h + 1) * DH].astype(jnp.bfloat16)
                s = lax.dot_general(q_bh, k_bh, (((1,), (1,)), ((), ())),
                                    preferred_element_type=jnp.float32)
                p = jnp.where(mask, jnp.exp(s * 0.125), 0.0)
                p16 = p.astype(jnp.bfloat16)
                lr = lax.dot_general(ones8, p16, (((1,), (1,)), ((), ())),
                                     preferred_element_type=jnp.float32)
                acc = jnp.dot(p16, v_bh, preferred_element_type=jnp.float32)
                part_ref[b, 0:SQ, h * DH:(h + 1) * DH] = acc
                part_ref[b, SQ + h:SQ + h + 1, :] = lr[0:1, :]

        for k in range(LOG2):
            partner = me ^ (1 << k)
            send_ref[...] = part_ref[...].astype(jnp.bfloat16)
            rdma = pltpu.make_async_remote_copy(
                src_ref=send_ref,
                dst_ref=recv_ref.at[k],
                send_sem=send_sems.at[k],
                recv_sem=recv_sems.at[k],
                device_id=(partner,),
                device_id_type=_DevT.MESH,
            )
            rdma.start()
            rdma.wait()
            part_ref[...] = part_ref[...] + recv_ref[k].astype(jnp.float32)

        wo = wo_ref[:, :].astype(jnp.bfloat16)
        i2d = lax.broadcasted_iota(jnp.int32, (SQ, SQ), 0)
        j2d = lax.broadcasted_iota(jnp.int32, (SQ, SQ), 1)
        eye = (i2d == j2d).astype(jnp.bfloat16)
        for b in range(B):
            l_rows = part_ref[b, SQ:SQ + 8, :].astype(jnp.bfloat16)
            lt = lax.dot_general(eye, l_rows, (((1,), (1,)), ((), ())),
                                 preferred_element_type=jnp.float32)
            denom = jnp.concatenate(
                [jnp.broadcast_to(lt[:, h:h + 1], (SQ, DH))
                 for h in range(HQ)], axis=1)
            ctx = part_ref[b, 0:SQ, :] / denom
            out_ref[b] = jnp.dot(ctx.astype(jnp.bfloat16), wo,
                                 preferred_element_type=jnp.float32)

    return pl.pallas_call(
        body,
        out_shape=jax.ShapeDtypeStruct((B, SQ, D_MODEL), jnp.float32),
        in_specs=[pl.BlockSpec(memory_space=pltpu.VMEM)] * 5,
        out_specs=pl.BlockSpec(memory_space=pltpu.VMEM),
        scratch_shapes=[
            pltpu.VMEM((B, ROWS, HD), jnp.float32),
            pltpu.VMEM((B, ROWS, HD), jnp.bfloat16),
            pltpu.VMEM((LOG2, B, ROWS, HD), jnp.bfloat16),
            pltpu.SemaphoreType.DMA((LOG2,)),
            pltpu.SemaphoreType.DMA((LOG2,)),
        ],
        compiler_params=pltpu.CompilerParams(collective_id=0),
    )(x, Wq, K2, V2, Wo)


# device time: 42559 ns/iter; 1.9129x vs baseline; 1.0223x over previous
import jax
import jax.numpy as jnp
from jax import lax
from jax.experimental import pallas as pl
from jax.experimental.pallas import tpu as pltpu

N_DEV = 32
LOG2 = 5
B, SQ, HQ, DH = 2, 256, 4, 64
D_MODEL = 512
HD = HQ * DH
SKV_LOC = 256
ROWS = SQ + 16

_DevT = getattr(pl, "DeviceIdType", None) or pltpu.DeviceIdType


def kernel(x, Wq, K_ext, V_ext, Wo):
    K2 = K_ext.reshape(B, SKV_LOC, HD)
    V2 = V_ext.reshape(B, SKV_LOC, HD)

    def body(x_ref, wq_ref, k_ref, v_ref, wo_ref, out_ref,
             part_ref, send_ref, recv_ref, send_sems, recv_sems):
        me = lax.axis_index("i")

        bar = pltpu.get_barrier_semaphore()
        for k in range(LOG2):
            pl.semaphore_signal(bar, inc=1, device_id=(me ^ (1 << k),),
                                device_id_type=_DevT.MESH)
        pl.semaphore_wait(bar, LOG2)

        wq = wq_ref[:, :].astype(jnp.bfloat16)
        rows = lax.broadcasted_iota(jnp.int32, (SQ, SKV_LOC), 0) // 64
        cols = lax.broadcasted_iota(jnp.int32, (SQ, SKV_LOC), 1) // 64
        mask = rows == cols
        ones8 = jnp.ones((8, SQ), jnp.bfloat16)

        def compute_partial(b):
            part_ref[b, SQ + HQ:ROWS, :] = jnp.zeros((ROWS - SQ - HQ, HD),
                                                     jnp.float32)
            q_b = jnp.dot(x_ref[b].astype(jnp.bfloat16), wq,
                          preferred_element_type=jnp.float32)
            q16 = q_b.astype(jnp.bfloat16)
            for h in range(HQ):
                q_bh = q16[:, h * DH:(h + 1) * DH]
                k_bh = k_ref[b][:, h * DH:(h + 1) * DH].astype(jnp.bfloat16)
                v_bh = v_ref[b][:, h * DH:(h + 1) * DH].astype(jnp.bfloat16)
                s = lax.dot_general(q_bh, k_bh, (((1,), (1,)), ((), ())),
                                    preferred_element_type=jnp.float32)
                p = jnp.where(mask, jnp.exp(s * 0.125), 0.0)
                p16 = p.astype(jnp.bfloat16)
                lr = lax.dot_general(ones8, p16, (((1,), (1,)), ((), ())),
                                     preferred_element_type=jnp.float32)
                acc = jnp.dot(p16, v_bh, preferred_element_type=jnp.float32)
                part_ref[b, 0:SQ, h * DH:(h + 1) * DH] = acc
                part_ref[b, SQ + h:SQ + h + 1, :] = lr[0:1, :]

        def make_rdma(k, j, partner):
            return pltpu.make_async_remote_copy(
                src_ref=send_ref.at[j],
                dst_ref=recv_ref.at[k, j],
                send_sem=send_sems.at[k, j],
                recv_sem=recv_sems.at[k, j],
                device_id=(partner,),
                device_id_type=_DevT.MESH,
            )

        rd_prev = [None, None]
        partner0 = me ^ 1
        compute_partial(0)
        send_ref[0] = part_ref[0].astype(jnp.bfloat16)
        rd0 = make_rdma(0, 0, partner0)
        rd0.start()
        compute_partial(1)
        send_ref[1] = part_ref[1].astype(jnp.bfloat16)
        rd1 = make_rdma(0, 1, partner0)
        rd1.start()
        rd_prev = [rd0, rd1]
        rd_prev[0].wait_recv()
        part_ref[0] = part_ref[0] + recv_ref[0, 0].astype(jnp.float32)
        rd_prev[1].wait_recv()
        part_ref[1] = part_ref[1] + recv_ref[0, 1].astype(jnp.float32)

        for k in range(1, LOG2):
            partner = me ^ (1 << k)
            rds = [None, None]
            for j in range(2):
                rd_prev[j].wait_send()
                send_ref[j] = part_ref[j].astype(jnp.bfloat16)
                rds[j] = make_rdma(k, j, partner)
                rds[j].start()
            for j in range(2):
                rds[j].wait_recv()
                part_ref[j] = part_ref[j] + recv_ref[k, j].astype(jnp.float32)
            rd_prev = rds

        rd_prev[0].wait_send()
        rd_prev[1].wait_send()

        wo = wo_ref[:, :].astype(jnp.bfloat16)
        i2d = lax.broadcasted_iota(jnp.int32, (SQ, SQ), 0)
        j2d = lax.broadcasted_iota(jnp.int32, (SQ, SQ), 1)
        eye = (i2d == j2d).astype(jnp.bfloat16)
        for b in range(B):
            l_rows = part_ref[b, SQ:SQ + 8, :].astype(jnp.bfloat16)
            lt = lax.dot_general(eye, l_rows, (((1,), (1,)), ((), ())),
                                 preferred_element_type=jnp.float32)
            denom = jnp.concatenate(
                [jnp.broadcast_to(lt[:, h:h + 1], (SQ, DH))
                 for h in range(HQ)], axis=1)
            ctx = part_ref[b, 0:SQ, :] / denom
            out_ref[b] = jnp.dot(ctx.astype(jnp.bfloat16), wo,
                                 preferred_element_type=jnp.float32)

    return pl.pallas_call(
        body,
        out_shape=jax.ShapeDtypeStruct((B, SQ, D_MODEL), jnp.float32),
        in_specs=[pl.BlockSpec(memory_space=pltpu.VMEM)] * 5,
        out_specs=pl.BlockSpec(memory_space=pltpu.VMEM),
        scratch_shapes=[
            pltpu.VMEM((B, ROWS, HD), jnp.float32),
            pltpu.VMEM((B, ROWS, HD), jnp.bfloat16),
            pltpu.VMEM((LOG2, B, ROWS, HD), jnp.bfloat16),
            pltpu.SemaphoreType.DMA((LOG2, 2)),
            pltpu.SemaphoreType.DMA((LOG2, 2)),
        ],
        compiler_params=pltpu.CompilerParams(collective_id=0),
    )(x, Wq, K2, V2, Wo)
